# baseline (speedup 1.0000x reference)
import functools

import jax
import jax.numpy as jnp
from jax import lax
from jax.experimental import pallas as pl
from jax.experimental.pallas import tpu as pltpu
from jax.experimental.pallas import tpu_sc as plsc

BATCH = 16384
EMBED_DIM = 32

_info = plsc.get_sparse_core_info()
_NC, _NS = _info.num_cores, _info.num_subcores
_NW = _NC * _NS
_B_PER_W = BATCH // _NW


_mesh = plsc.VectorSubcoreMesh(core_axis_name="c", subcore_axis_name="s")


@functools.partial(
    pl.kernel,
    mesh=_mesh,
    out_type=(
        jax.ShapeDtypeStruct((BATCH, EMBED_DIM), jnp.float32),
        jax.ShapeDtypeStruct((BATCH, EMBED_DIM), jnp.float32),
    ),
    scratch_types=[
        pltpu.VMEM((_B_PER_W,), jnp.int32),
        pltpu.VMEM((_B_PER_W,), jnp.int32),
        pltpu.VMEM((_B_PER_W, EMBED_DIM), jnp.float32),
        pltpu.VMEM((_B_PER_W, EMBED_DIM), jnp.float32),
        pltpu.SemaphoreType.DMA,
        pltpu.SemaphoreType.DMA,
    ],
    compiler_params=pltpu.CompilerParams(use_tc_tiling_on_sc=False),
)
def _gather2(
    user_idx_hbm,
    item_idx_hbm,
    user_tab_hbm,
    item_tab_hbm,
    user_out_hbm,
    item_out_hbm,
    uidx_v,
    iidx_v,
    urows_v,
    irows_v,
    usem,
    isem,
):
    wid = lax.axis_index("s") * _NC + lax.axis_index("c")
    base = wid * _B_PER_W
    pltpu.sync_copy(user_idx_hbm.at[pl.ds(base, _B_PER_W)], uidx_v)
    pltpu.sync_copy(item_idx_hbm.at[pl.ds(base, _B_PER_W)], iidx_v)
    ucopy = pltpu.async_copy(user_tab_hbm.at[uidx_v], urows_v, usem)
    icopy = pltpu.async_copy(item_tab_hbm.at[iidx_v], irows_v, isem)
    ucopy.wait()
    icopy.wait()
    pltpu.sync_copy(urows_v, user_out_hbm.at[pl.ds(base, _B_PER_W)])
    pltpu.sync_copy(irows_v, item_out_hbm.at[pl.ds(base, _B_PER_W)])


@jax.jit
def kernel(user_input, item_input, user_table, item_table):
    return _gather2(
        user_input.astype(jnp.int32),
        item_input.astype(jnp.int32),
        user_table,
        item_table,
    )


# speedup vs baseline: 3.2773x; 3.2773x over previous
import functools

import jax
import jax.numpy as jnp
from jax import lax
from jax.experimental import pallas as pl
from jax.experimental.pallas import tpu as pltpu
from jax.experimental.pallas import tpu_sc as plsc

BATCH = 16384
EMBED_DIM = 32
NUM_ROWS = 1000000

_info = plsc.get_sparse_core_info()
_NC, _NS = _info.num_cores, _info.num_subcores
_NW = _NC * _NS
_B_PER_W = BATCH // _NW
_CHUNK = 8

_mesh = plsc.VectorSubcoreMesh(core_axis_name="c", subcore_axis_name="s")


@functools.partial(
    pl.kernel,
    mesh=_mesh,
    out_type=(
        jax.ShapeDtypeStruct((EMBED_DIM, BATCH), jnp.float32),
        jax.ShapeDtypeStruct((EMBED_DIM, BATCH), jnp.float32),
    ),
    scratch_types=[
        pltpu.VMEM((_B_PER_W,), jnp.int32),
        pltpu.VMEM((_B_PER_W,), jnp.int32),
        pltpu.VMEM((_CHUNK, EMBED_DIM, 128), jnp.float32),
        pltpu.VMEM((_CHUNK, EMBED_DIM, 128), jnp.float32),
        pltpu.VMEM((EMBED_DIM, _B_PER_W), jnp.float32),
        pltpu.VMEM((EMBED_DIM, _B_PER_W), jnp.float32),
        pltpu.SemaphoreType.DMA,
        pltpu.SemaphoreType.DMA,
    ],
    compiler_params=pltpu.CompilerParams(needs_layout_passes=False),
)
def _gather2(
    user_idx_hbm,
    item_idx_hbm,
    user_tab_hbm,
    item_tab_hbm,
    user_out_hbm,
    item_out_hbm,
    uidx_v,
    iidx_v,
    ublk_v,
    iblk_v,
    uvals_v,
    ivals_v,
    usem,
    isem,
):
    wid = lax.axis_index("s") * _NC + lax.axis_index("c")
    base = wid * _B_PER_W
    pltpu.sync_copy(user_idx_hbm.at[pl.ds(base, _B_PER_W)], uidx_v)
    pltpu.sync_copy(item_idx_hbm.at[pl.ds(base, _B_PER_W)], iidx_v)

    j_lo = lax.iota(jnp.int32, 16)
    j_hi = j_lo + 16

    def fire(tab_hbm, offv, blk_v, sem, h):
        copies = []
        for c in range(_CHUNK):
            off = pl.multiple_of(offv[h * _CHUNK + c], 128)
            copies.append(
                pltpu.async_copy(
                    tab_hbm.at[:, pl.ds(off, 128)], blk_v.at[c], sem
                )
            )
        return copies

    def extract(lanev, blk_v, vals_v, g, h):
        for c in range(_CHUNK):
            b = g * 16 + h * _CHUNK + c
            lane16 = jnp.full((16,), lanev[h * _CHUNK + c], jnp.int32)
            b16 = jnp.full((16,), b, jnp.int32)
            lo = plsc.load_gather(blk_v.at[c], [j_lo, lane16])
            hi = plsc.load_gather(blk_v.at[c], [j_hi, lane16])
            plsc.store_scatter(vals_v, [j_lo, b16], lo)
            plsc.store_scatter(vals_v, [j_hi, b16], hi)

    def body(g, _):
        ui16 = uidx_v[pl.ds(g * 16, 16)]
        ii16 = iidx_v[pl.ds(g * 16, 16)]
        uoff = lax.shift_left(lax.shift_right_logical(ui16, 7), 7)
        ioff = lax.shift_left(lax.shift_right_logical(ii16, 7), 7)
        ulane = lax.rem(ui16, 128)
        ilane = lax.rem(ii16, 128)
        for h in range(2):
            ucopies = fire(user_tab_hbm, uoff, ublk_v, usem, h)
            icopies = fire(item_tab_hbm, ioff, iblk_v, isem, h)
            for cp in ucopies:
                cp.wait()
            extract(ulane, ublk_v, uvals_v, g, h)
            for cp in icopies:
                cp.wait()
            extract(ilane, iblk_v, ivals_v, g, h)
        return 0

    lax.fori_loop(0, _B_PER_W // 16, body, 0)

    pltpu.sync_copy(uvals_v, user_out_hbm.at[:, pl.ds(base, _B_PER_W)])
    pltpu.sync_copy(ivals_v, item_out_hbm.at[:, pl.ds(base, _B_PER_W)])


@jax.jit
def kernel(user_input, item_input, user_table, item_table):
    u, it = _gather2(
        user_input.astype(jnp.int32),
        item_input.astype(jnp.int32),
        user_table.T,
        item_table.T,
    )
    return u.T, it.T
